# baseline (speedup 1.0000x reference)
import functools

import jax
import jax.numpy as jnp
from jax import lax
from jax.experimental import pallas as pl
from jax.experimental.pallas import tpu as pltpu
from jax.experimental.pallas import tpu_sc as plsc

N = 10000
E = 320000
D = 128
H = 256

NUM_TILES = 32
CHUNK = 128
NCH = 80
NPH = 2
PCH = NCH // NPH
EDGES_PER_TILE = CHUNK * NCH
EPAD = NUM_TILES * EDGES_PER_TILE
NPAD = 10112
ZROWS = NPAD // 16
NBUF = 2


def _sc_body(x_hbm, send_hbm, recv_hbm, zeros_hbm, out_hbm,
             agg_sh, send_v, recv_v, rows0, rows1, sem0, sem1):
    c = lax.axis_index("c")
    s = lax.axis_index("s")
    wid = c * 16 + s
    rows = (rows0, rows1)
    sems = (sem0, sem1)

    pltpu.sync_copy(zeros_hbm, agg_sh.at[pl.ds(s * ZROWS, ZROWS)])
    plsc.subcore_barrier()

    def gather(j, b):
        return pltpu.make_async_copy(x_hbm.at[send_v.at[j]], rows[b], sems[b])

    for h in range(NPH):
        pltpu.sync_copy(send_hbm.at[wid, pl.ds(h * PCH, PCH)], send_v)
        pltpu.sync_copy(recv_hbm.at[wid, pl.ds(h * PCH, PCH)], recv_v)
        for b in range(NBUF):
            gather(b, b).start()

        def window(w, carry):
            i0 = w * NBUF
            for b in range(NBUF):
                j = i0 + b
                gather(j, b).wait()
                pltpu.sync_copy(rows[b], agg_sh.at[recv_v.at[j]], add=True)

                @pl.when(j + NBUF < PCH)
                def _():
                    gather(j + NBUF, b).start()

            return carry

        lax.fori_loop(0, PCH // NBUF, window, 0)

    plsc.subcore_barrier()

    r0 = s * ZROWS
    pltpu.sync_copy(agg_sh.at[pl.ds(r0, ZROWS)],
                    out_hbm.at[c, pl.ds(r0, ZROWS)])


_sc_aggregate = functools.partial(
    pl.kernel,
    out_type=jax.ShapeDtypeStruct((2, NPAD, D), jnp.float32),
    mesh=plsc.VectorSubcoreMesh(core_axis_name="c", subcore_axis_name="s"),
    scratch_types=[
        pltpu.VMEM_SHARED((NPAD, D), jnp.float32),
        pltpu.VMEM((PCH, CHUNK), jnp.int32),
        pltpu.VMEM((PCH, CHUNK), jnp.int32),
    ]
    + [pltpu.VMEM((CHUNK, D), jnp.float32)] * NBUF
    + [pltpu.SemaphoreType.DMA] * NBUF,
)(_sc_body)


def _mlp_body(x_ref, p_ref, w1_ref, b1_ref, w2_ref, b2_ref, o_ref):
    agg = p_ref[0] + p_ref[1]
    h = (
        jnp.dot(x_ref[...], w1_ref[:D, :], preferred_element_type=jnp.float32)
        + jnp.dot(agg, w1_ref[D:, :], preferred_element_type=jnp.float32)
        + b1_ref[...]
    )
    h = jnp.maximum(h, 0.0)
    o_ref[...] = (
        jnp.dot(h, w2_ref[...], preferred_element_type=jnp.float32)
        + b2_ref[...]
    )


def _mlp(x, partials, W1, b1, W2, b2):
    blk = 2000
    grid = (N // blk,)
    return pl.pallas_call(
        _mlp_body,
        grid=grid,
        in_specs=[
            pl.BlockSpec((blk, D), lambda i: (i, 0)),
            pl.BlockSpec((2, blk, D), lambda i: (0, i, 0)),
            pl.BlockSpec((2 * D, H), lambda i: (0, 0)),
            pl.BlockSpec((1, H), lambda i: (0, 0)),
            pl.BlockSpec((H, D), lambda i: (0, 0)),
            pl.BlockSpec((1, D), lambda i: (0, 0)),
        ],
        out_specs=pl.BlockSpec((blk, D), lambda i: (i, 0)),
        out_shape=jax.ShapeDtypeStruct((N, D), jnp.float32),
    )(x, partials, W1, b1, W2, b2)


def kernel(x, edge_index, W1, b1, W2, b2):
    senders = edge_index[0]
    receivers = edge_index[1]
    pad = EPAD - E
    senders_p = jnp.concatenate(
        [senders, jnp.zeros((pad,), dtype=jnp.int32)]).reshape(
            NUM_TILES, NCH, CHUNK)
    receivers_p = jnp.concatenate(
        [receivers, jnp.full((pad,), N, dtype=jnp.int32)]).reshape(
            NUM_TILES, NCH, CHUNK)
    zeros = jnp.zeros((ZROWS, D), dtype=jnp.float32)
    partials = _sc_aggregate(x, senders_p, receivers_p, zeros)[:, :N]
    return _mlp(x, partials, W1, b1.reshape(1, H), W2, b2.reshape(1, D))


# speedup vs baseline: 2.3038x; 2.3038x over previous
import functools

import jax
import jax.numpy as jnp
from jax import lax
from jax.experimental import pallas as pl
from jax.experimental.pallas import tpu as pltpu
from jax.experimental.pallas import tpu_sc as plsc

N = 10000
E = 320000
D = 128
H = 256
DH = D // 2

NUM_TILES = 16
CHUNK = 128
NCH = 160
NPH = 4
PCH = NCH // NPH
EDGES_PER_TILE = CHUNK * NCH
EPAD = NUM_TILES * EDGES_PER_TILE
NPAD = 10112
ZROWS = NPAD // 16
NBUF = 2


def _sc_body(xs_hbm, send_hbm, recv_hbm, zeros_hbm, out_hbm,
             x_sh, agg_sh, send_v, recv_v, rows0, rows1, sem0, sem1):
    c = lax.axis_index("c")
    s = lax.axis_index("s")
    rows = (rows0, rows1)
    sems = (sem0, sem1)

    @pl.when(s < 15)
    def _():
        pltpu.sync_copy(xs_hbm.at[c, pl.ds(s * 632, 632)],
                        x_sh.at[pl.ds(s * 632, 632)])

    @pl.when(s == 15)
    def _():
        pltpu.sync_copy(xs_hbm.at[c, pl.ds(9480, 520)],
                        x_sh.at[pl.ds(9480, 520)])

    pltpu.sync_copy(zeros_hbm, agg_sh.at[pl.ds(s * ZROWS, ZROWS)])
    plsc.subcore_barrier()

    def gather(j, b):
        return pltpu.make_async_copy(x_sh.at[send_v.at[j]], rows[b], sems[b])

    for h in range(NPH):
        pltpu.sync_copy(send_hbm.at[s, pl.ds(h * PCH, PCH)], send_v)
        pltpu.sync_copy(recv_hbm.at[s, pl.ds(h * PCH, PCH)], recv_v)
        for b in range(NBUF):
            gather(b, b).start()

        def window(w, carry):
            i0 = w * NBUF
            for b in range(NBUF):
                j = i0 + b
                gather(j, b).wait()
                pltpu.sync_copy(rows[b], agg_sh.at[recv_v.at[j]], add=True)

                @pl.when(j + NBUF < PCH)
                def _():
                    gather(j + NBUF, b).start()

            return carry

        lax.fori_loop(0, PCH // NBUF, window, 0)

    plsc.subcore_barrier()

    r0 = s * ZROWS
    pltpu.sync_copy(agg_sh.at[pl.ds(r0, ZROWS)],
                    out_hbm.at[c, pl.ds(r0, ZROWS)])


_sc_aggregate = functools.partial(
    pl.kernel,
    out_type=jax.ShapeDtypeStruct((2, NPAD, DH), jnp.float32),
    mesh=plsc.VectorSubcoreMesh(core_axis_name="c", subcore_axis_name="s"),
    scratch_types=[
        pltpu.VMEM_SHARED((N, DH), jnp.float32),
        pltpu.VMEM_SHARED((NPAD, DH), jnp.float32),
        pltpu.VMEM((PCH, CHUNK), jnp.int32),
        pltpu.VMEM((PCH, CHUNK), jnp.int32),
    ]
    + [pltpu.VMEM((CHUNK, DH), jnp.float32)] * NBUF
    + [pltpu.SemaphoreType.DMA] * NBUF,
)(_sc_body)


def _mlp_body(x_ref, p_ref, w1_ref, b1_ref, w2_ref, b2_ref, o_ref):
    h = (
        jnp.dot(x_ref[...], w1_ref[:D, :], preferred_element_type=jnp.float32)
        + jnp.dot(p_ref[0], w1_ref[D:D + DH, :],
                  preferred_element_type=jnp.float32)
        + jnp.dot(p_ref[1], w1_ref[D + DH:, :],
                  preferred_element_type=jnp.float32)
        + b1_ref[...]
    )
    h = jnp.maximum(h, 0.0)
    o_ref[...] = (
        jnp.dot(h, w2_ref[...], preferred_element_type=jnp.float32)
        + b2_ref[...]
    )


def _mlp(x, partials, W1, b1, W2, b2):
    blk = 2000
    grid = (N // blk,)
    return pl.pallas_call(
        _mlp_body,
        grid=grid,
        in_specs=[
            pl.BlockSpec((blk, D), lambda i: (i, 0)),
            pl.BlockSpec((2, blk, DH), lambda i: (0, i, 0)),
            pl.BlockSpec((2 * D, H), lambda i: (0, 0)),
            pl.BlockSpec((1, H), lambda i: (0, 0)),
            pl.BlockSpec((H, D), lambda i: (0, 0)),
            pl.BlockSpec((1, D), lambda i: (0, 0)),
        ],
        out_specs=pl.BlockSpec((blk, D), lambda i: (i, 0)),
        out_shape=jax.ShapeDtypeStruct((N, D), jnp.float32),
    )(x, partials, W1, b1, W2, b2)


def kernel(x, edge_index, W1, b1, W2, b2):
    senders = edge_index[0]
    receivers = edge_index[1]
    pad = EPAD - E
    senders_p = jnp.concatenate(
        [senders, jnp.zeros((pad,), dtype=jnp.int32)]).reshape(
            NUM_TILES, NCH, CHUNK)
    receivers_p = jnp.concatenate(
        [receivers, jnp.full((pad,), N, dtype=jnp.int32)]).reshape(
            NUM_TILES, NCH, CHUNK)
    zeros = jnp.zeros((ZROWS, DH), dtype=jnp.float32)
    xs = jnp.stack([x[:, :DH], x[:, DH:]])
    partials = _sc_aggregate(xs, senders_p, receivers_p, zeros)[:, :N]
    return _mlp(x, partials, W1, b1.reshape(1, H), W2, b2.reshape(1, D))


# speedup vs baseline: 2.3765x; 1.0315x over previous
import functools

import jax
import jax.numpy as jnp
from jax import lax
from jax.experimental import pallas as pl
from jax.experimental.pallas import tpu as pltpu
from jax.experimental.pallas import tpu_sc as plsc

N = 10000
E = 320000
D = 128
H = 256
DH = D // 2

NUM_TILES = 16
CHUNK = 128
NCH = 160
NPH = 4
PCH = NCH // NPH
EDGES_PER_TILE = CHUNK * NCH
EPAD = NUM_TILES * EDGES_PER_TILE
NPAD = 10112
ZROWS = NPAD // 16
NBUF = 2


def _sc_body(xs_hbm, send_hbm, recv_hbm, zeros_hbm, out_hbm,
             x_sh, agg_sh, send_v, recv_v, rows0, rows1, sem0, sem1):
    c = lax.axis_index("c")
    s = lax.axis_index("s")
    rows = (rows0, rows1)
    sems = (sem0, sem1)

    @pl.when(s < 15)
    def _():
        pltpu.sync_copy(xs_hbm.at[c, pl.ds(s * 632, 632)],
                        x_sh.at[pl.ds(s * 632, 632)])

    @pl.when(s == 15)
    def _():
        pltpu.sync_copy(xs_hbm.at[c, pl.ds(9480, 520)],
                        x_sh.at[pl.ds(9480, 520)])

    pltpu.sync_copy(zeros_hbm, agg_sh.at[pl.ds(s * ZROWS, ZROWS)])
    pltpu.sync_copy(send_hbm.at[s, pl.ds(0, PCH)], send_v)
    pltpu.sync_copy(recv_hbm.at[s, pl.ds(0, PCH)], recv_v)
    plsc.subcore_barrier()

    def gather(j, b):
        return pltpu.make_async_copy(x_sh.at[send_v.at[j]], rows[b], sems[b])

    def scatter(j, b):
        pltpu.sync_copy(rows[b], agg_sh.at[recv_v.at[j]], add=True)

    for h in range(NPH):
        if h > 0:
            pltpu.sync_copy(send_hbm.at[s, pl.ds(h * PCH, PCH)], send_v)
            pltpu.sync_copy(recv_hbm.at[s, pl.ds(h * PCH, PCH)], recv_v)
        for b in range(NBUF):
            gather(b, b).start()

        def window(w, carry):
            i0 = w * NBUF
            for b in range(NBUF):
                j = i0 + b
                gather(j, b).wait()
                scatter(j, b)
                gather(j + NBUF, b).start()
            return carry

        lax.fori_loop(0, PCH // NBUF - 1, window, 0)
        for b in range(NBUF):
            j = PCH - NBUF + b
            gather(j, b).wait()
            scatter(j, b)

    plsc.subcore_barrier()

    r0 = s * ZROWS
    pltpu.sync_copy(agg_sh.at[pl.ds(r0, ZROWS)],
                    out_hbm.at[c, pl.ds(r0, ZROWS)])


_sc_aggregate = functools.partial(
    pl.kernel,
    out_type=jax.ShapeDtypeStruct((2, NPAD, DH), jnp.float32),
    mesh=plsc.VectorSubcoreMesh(core_axis_name="c", subcore_axis_name="s"),
    scratch_types=[
        pltpu.VMEM_SHARED((N, DH), jnp.float32),
        pltpu.VMEM_SHARED((NPAD, DH), jnp.float32),
        pltpu.VMEM((PCH, CHUNK), jnp.int32),
        pltpu.VMEM((PCH, CHUNK), jnp.int32),
    ]
    + [pltpu.VMEM((CHUNK, DH), jnp.float32)] * NBUF
    + [pltpu.SemaphoreType.DMA] * NBUF,
)(_sc_body)


def _mlp_body(x_ref, p_ref, w1_ref, b1_ref, w2_ref, b2_ref, o_ref):
    h = (
        jnp.dot(x_ref[...], w1_ref[:D, :], preferred_element_type=jnp.float32)
        + jnp.dot(p_ref[0], w1_ref[D:D + DH, :],
                  preferred_element_type=jnp.float32)
        + jnp.dot(p_ref[1], w1_ref[D + DH:, :],
                  preferred_element_type=jnp.float32)
        + b1_ref[...]
    )
    h = jnp.maximum(h, 0.0)
    o_ref[...] = (
        jnp.dot(h, w2_ref[...], preferred_element_type=jnp.float32)
        + b2_ref[...]
    )


def _mlp(x, partials, W1, b1, W2, b2):
    blk = 2000
    grid = (N // blk,)
    return pl.pallas_call(
        _mlp_body,
        grid=grid,
        in_specs=[
            pl.BlockSpec((blk, D), lambda i: (i, 0)),
            pl.BlockSpec((2, blk, DH), lambda i: (0, i, 0)),
            pl.BlockSpec((2 * D, H), lambda i: (0, 0)),
            pl.BlockSpec((1, H), lambda i: (0, 0)),
            pl.BlockSpec((H, D), lambda i: (0, 0)),
            pl.BlockSpec((1, D), lambda i: (0, 0)),
        ],
        out_specs=pl.BlockSpec((blk, D), lambda i: (i, 0)),
        out_shape=jax.ShapeDtypeStruct((N, D), jnp.float32),
    )(x, partials, W1, b1, W2, b2)


def kernel(x, edge_index, W1, b1, W2, b2):
    senders = edge_index[0]
    receivers = edge_index[1]
    pad = EPAD - E
    senders_p = jnp.concatenate(
        [senders, jnp.zeros((pad,), dtype=jnp.int32)]).reshape(
            NUM_TILES, NCH, CHUNK)
    receivers_p = jnp.concatenate(
        [receivers, jnp.full((pad,), N, dtype=jnp.int32)]).reshape(
            NUM_TILES, NCH, CHUNK)
    zeros = jnp.zeros((ZROWS, DH), dtype=jnp.float32)
    xs = jnp.stack([x[:, :DH], x[:, DH:]])
    partials = _sc_aggregate(xs, senders_p, receivers_p, zeros)
    return _mlp(x, partials, W1, b1.reshape(1, H), W2, b2.reshape(1, D))


# speedup vs baseline: 2.5173x; 1.0592x over previous
import functools

import jax
import jax.numpy as jnp
from jax import lax
from jax.experimental import pallas as pl
from jax.experimental.pallas import tpu as pltpu
from jax.experimental.pallas import tpu_sc as plsc

N = 10000
E = 320000
D = 128
H = 256
DH = D // 2

NUM_TILES = 16
CHUNK = 128
NCHT = 2504
PCH = 40
NPAD = 10112
ZROWS = NPAD // 16
NBUF = 2


def _sc_body(xs_hbm, eidx_hbm, zeros_hbm, out_hbm,
             x_sh, agg_sh, send_v, recv_v, rows0, rows1, sem0, sem1):
    c = lax.axis_index("c")
    s = lax.axis_index("s")
    rows = (rows0, rows1)
    sems = (sem0, sem1)
    base = jnp.minimum(s * 160, 2400)

    @pl.when(s < 15)
    def _():
        pltpu.sync_copy(xs_hbm.at[c, pl.ds(s * 632, 632)],
                        x_sh.at[pl.ds(s * 632, 632)])

    @pl.when(s == 15)
    def _():
        pltpu.sync_copy(xs_hbm.at[c, pl.ds(9480, 520)],
                        x_sh.at[pl.ds(9480, 520)])

    pltpu.sync_copy(zeros_hbm, agg_sh.at[pl.ds(s * ZROWS, ZROWS)])
    pltpu.sync_copy(eidx_hbm.at[0, pl.ds(base, PCH)], send_v)
    pltpu.sync_copy(eidx_hbm.at[1, pl.ds(base, PCH)], recv_v)
    plsc.subcore_barrier()

    def gather(j, b):
        return pltpu.make_async_copy(x_sh.at[send_v.at[j]], rows[b], sems[b])

    def scatter(j, b):
        pltpu.sync_copy(rows[b], agg_sh.at[recv_v.at[j]], add=True)

    def run_phase(off, cnt):
        if off is not None:
            pltpu.sync_copy(eidx_hbm.at[0, pl.ds(off, cnt)],
                            send_v.at[pl.ds(0, cnt)])
            pltpu.sync_copy(eidx_hbm.at[1, pl.ds(off, cnt)],
                            recv_v.at[pl.ds(0, cnt)])
        for b in range(NBUF):
            gather(b, b).start()

        def window(w, carry):
            i0 = w * NBUF
            for b in range(NBUF):
                j = i0 + b
                gather(j, b).wait()
                scatter(j, b)
                gather(j + NBUF, b).start()
            return carry

        lax.fori_loop(0, cnt // NBUF - 1, window, 0)
        for b in range(NBUF):
            j = cnt - NBUF + b
            gather(j, b).wait()
            scatter(j, b)

    @pl.when(s < 15)
    def _():
        run_phase(None, PCH)
        for h in range(1, 4):
            run_phase(base + h * PCH, PCH)

    @pl.when(s == 15)
    def _():
        run_phase(None, PCH)
        run_phase(base + PCH, PCH)
        run_phase(base + 2 * PCH, 24)

    plsc.subcore_barrier()

    r0 = s * ZROWS
    pltpu.sync_copy(agg_sh.at[pl.ds(r0, ZROWS)],
                    out_hbm.at[c, pl.ds(r0, ZROWS)])


_sc_aggregate = functools.partial(
    pl.kernel,
    out_type=jax.ShapeDtypeStruct((2, NPAD, DH), jnp.float32),
    mesh=plsc.VectorSubcoreMesh(core_axis_name="c", subcore_axis_name="s"),
    scratch_types=[
        pltpu.VMEM_SHARED((N, DH), jnp.float32),
        pltpu.VMEM_SHARED((NPAD, DH), jnp.float32),
        pltpu.VMEM((PCH, CHUNK), jnp.int32),
        pltpu.VMEM((PCH, CHUNK), jnp.int32),
    ]
    + [pltpu.VMEM((CHUNK, DH), jnp.float32)] * NBUF
    + [pltpu.SemaphoreType.DMA] * NBUF,
)(_sc_body)


def _mlp_body(x_ref, p_ref, w1_ref, b1_ref, w2_ref, b2_ref, o_ref):
    h = (
        jnp.dot(x_ref[...], w1_ref[:D, :], preferred_element_type=jnp.float32)
        + jnp.dot(p_ref[0], w1_ref[D:D + DH, :],
                  preferred_element_type=jnp.float32)
        + jnp.dot(p_ref[1], w1_ref[D + DH:, :],
                  preferred_element_type=jnp.float32)
        + b1_ref[...]
    )
    h = jnp.maximum(h, 0.0)
    o_ref[...] = (
        jnp.dot(h, w2_ref[...], preferred_element_type=jnp.float32)
        + b2_ref[...]
    )


def _mlp(x, partials, W1, b1, W2, b2):
    blk = 2000
    grid = (N // blk,)
    return pl.pallas_call(
        _mlp_body,
        grid=grid,
        in_specs=[
            pl.BlockSpec((blk, D), lambda i: (i, 0)),
            pl.BlockSpec((2, blk, DH), lambda i: (0, i, 0)),
            pl.BlockSpec((2 * D, H), lambda i: (0, 0)),
            pl.BlockSpec((1, H), lambda i: (0, 0)),
            pl.BlockSpec((H, D), lambda i: (0, 0)),
            pl.BlockSpec((1, D), lambda i: (0, 0)),
        ],
        out_specs=pl.BlockSpec((blk, D), lambda i: (i, 0)),
        out_shape=jax.ShapeDtypeStruct((N, D), jnp.float32),
    )(x, partials, W1, b1, W2, b2)


def kernel(x, edge_index, W1, b1, W2, b2):
    padblk = jnp.concatenate([
        jnp.zeros((1, 512), jnp.int32),
        jnp.full((1, 512), N, jnp.int32),
    ])
    eidx = jnp.concatenate([edge_index, padblk], axis=1).reshape(
        2, NCHT, CHUNK)
    zeros = jnp.zeros((ZROWS, DH), dtype=jnp.float32)
    xs = jnp.stack([x[:, :DH], x[:, DH:]])
    partials = _sc_aggregate(xs, eidx, zeros)
    return _mlp(x, partials, W1, b1.reshape(1, H), W2, b2.reshape(1, D))


# speedup vs baseline: 2.5442x; 1.0107x over previous
import functools

import jax
import jax.numpy as jnp
from jax import lax
from jax.experimental import pallas as pl
from jax.experimental.pallas import tpu as pltpu
from jax.experimental.pallas import tpu_sc as plsc

N = 10000
E = 320000
D = 128
H = 256
DH = D // 2

NUM_TILES = 16
CHUNK = 128
NCHT = 2504
PCH = 32
NPAD = 10112
ZROWS = NPAD // 16
NBUF = 2


def _sc_body(xs_hbm, eidx_hbm, zeros_hbm, out_hbm,
             x_sh, agg_sh, send0, recv0, send1, recv1,
             rows0, rows1, sem0, sem1, isem0, isem1):
    c = lax.axis_index("c")
    s = lax.axis_index("s")
    rows = (rows0, rows1)
    sems = (sem0, sem1)
    idx = ((send0, recv0), (send1, recv1))
    isems = (isem0, isem1)
    base = jnp.minimum(s * 160, 2400)

    @pl.when(s < 15)
    def _():
        pltpu.sync_copy(xs_hbm.at[c, pl.ds(s * 632, 632)],
                        x_sh.at[pl.ds(s * 632, 632)])

    @pl.when(s == 15)
    def _():
        pltpu.sync_copy(xs_hbm.at[c, pl.ds(9480, 520)],
                        x_sh.at[pl.ds(9480, 520)])

    pltpu.sync_copy(zeros_hbm, agg_sh.at[pl.ds(s * ZROWS, ZROWS)])
    pltpu.sync_copy(eidx_hbm.at[0, pl.ds(base, PCH)], send0)
    pltpu.sync_copy(eidx_hbm.at[1, pl.ds(base, PCH)], recv0)
    plsc.subcore_barrier()

    def gather(send_v, j, b):
        return pltpu.make_async_copy(x_sh.at[send_v.at[j]], rows[b], sems[b])

    def scatter(recv_v, j, b):
        pltpu.sync_copy(rows[b], agg_sh.at[recv_v.at[j]], add=True)

    def prefetch(p, off, cnt):
        send_v, recv_v = idx[p]
        pltpu.make_async_copy(eidx_hbm.at[0, pl.ds(off, cnt)],
                              send_v.at[pl.ds(0, cnt)], isems[p]).start()
        pltpu.make_async_copy(eidx_hbm.at[1, pl.ds(off, cnt)],
                              recv_v.at[pl.ds(0, cnt)], isems[p]).start()

    def wait_prefetch(p, off, cnt):
        send_v, recv_v = idx[p]
        pltpu.make_async_copy(eidx_hbm.at[0, pl.ds(off, cnt)],
                              send_v.at[pl.ds(0, cnt)], isems[p]).wait()
        pltpu.make_async_copy(eidx_hbm.at[1, pl.ds(off, cnt)],
                              recv_v.at[pl.ds(0, cnt)], isems[p]).wait()

    def run_phases(plan):
        for ph, (off, cnt) in enumerate(plan):
            send_v, recv_v = idx[ph % 2]
            if ph > 0:
                wait_prefetch(ph % 2, off, cnt)
            if ph + 1 < len(plan):
                prefetch((ph + 1) % 2, plan[ph + 1][0], plan[ph + 1][1])
            for b in range(NBUF):
                gather(send_v, b, b).start()

            def window(w, carry, send_v=send_v, recv_v=recv_v):
                i0 = w * NBUF
                for b in range(NBUF):
                    j = i0 + b
                    gather(send_v, j, b).wait()
                    scatter(recv_v, j, b)
                    gather(send_v, j + NBUF, b).start()
                return carry

            lax.fori_loop(0, cnt // NBUF - 1, window, 0)
            for b in range(NBUF):
                j = cnt - NBUF + b
                gather(send_v, j, b).wait()
                scatter(recv_v, j, b)

    @pl.when(s < 15)
    def _():
        run_phases([(base + h * PCH, PCH) for h in range(5)])

    @pl.when(s == 15)
    def _():
        run_phases([(base, PCH), (base + PCH, PCH),
                    (base + 2 * PCH, PCH), (base + 3 * PCH, 8)])

    plsc.subcore_barrier()

    r0 = s * ZROWS
    pltpu.sync_copy(agg_sh.at[pl.ds(r0, ZROWS)],
                    out_hbm.at[c, pl.ds(r0, ZROWS)])


_sc_aggregate = functools.partial(
    pl.kernel,
    out_type=jax.ShapeDtypeStruct((2, NPAD, DH), jnp.float32),
    mesh=plsc.VectorSubcoreMesh(core_axis_name="c", subcore_axis_name="s"),
    scratch_types=[
        pltpu.VMEM_SHARED((N, DH), jnp.float32),
        pltpu.VMEM_SHARED((NPAD, DH), jnp.float32),
    ]
    + [pltpu.VMEM((PCH, CHUNK), jnp.int32)] * 4
    + [pltpu.VMEM((CHUNK, DH), jnp.float32)] * NBUF
    + [pltpu.SemaphoreType.DMA] * (NBUF + 2),
)(_sc_body)


def _mlp_body(x_ref, p_ref, w1_ref, b1_ref, w2_ref, b2_ref, o_ref):
    h = (
        jnp.dot(x_ref[...], w1_ref[:D, :], preferred_element_type=jnp.float32)
        + jnp.dot(p_ref[0], w1_ref[D:D + DH, :],
                  preferred_element_type=jnp.float32)
        + jnp.dot(p_ref[1], w1_ref[D + DH:, :],
                  preferred_element_type=jnp.float32)
        + b1_ref[...]
    )
    h = jnp.maximum(h, 0.0)
    o_ref[...] = (
        jnp.dot(h, w2_ref[...], preferred_element_type=jnp.float32)
        + b2_ref[...]
    )


def _mlp(x, partials, W1, b1, W2, b2):
    blk = 2000
    grid = (N // blk,)
    return pl.pallas_call(
        _mlp_body,
        grid=grid,
        in_specs=[
            pl.BlockSpec((blk, D), lambda i: (i, 0)),
            pl.BlockSpec((2, blk, DH), lambda i: (0, i, 0)),
            pl.BlockSpec((2 * D, H), lambda i: (0, 0)),
            pl.BlockSpec((1, H), lambda i: (0, 0)),
            pl.BlockSpec((H, D), lambda i: (0, 0)),
            pl.BlockSpec((1, D), lambda i: (0, 0)),
        ],
        out_specs=pl.BlockSpec((blk, D), lambda i: (i, 0)),
        out_shape=jax.ShapeDtypeStruct((N, D), jnp.float32),
    )(x, partials, W1, b1, W2, b2)


def kernel(x, edge_index, W1, b1, W2, b2):
    padblk = jnp.concatenate([
        jnp.zeros((1, 512), jnp.int32),
        jnp.full((1, 512), N, jnp.int32),
    ])
    eidx = jnp.concatenate([edge_index, padblk], axis=1).reshape(
        2, NCHT, CHUNK)
    zeros = jnp.zeros((ZROWS, DH), dtype=jnp.float32)
    xs = jnp.stack([x[:, :DH], x[:, DH:]])
    partials = _sc_aggregate(xs, eidx, zeros)
    return _mlp(x, partials, W1, b1.reshape(1, H), W2, b2.reshape(1, D))
